# baseline (speedup 1.0000x reference)
import functools

import jax
import jax.numpy as jnp
from jax import lax
from jax.experimental import pallas as pl
from jax.experimental.pallas import tpu as pltpu
from jax.experimental.pallas import tpu_sc as plsc

_NE = 64
_K = 8
_ROWS = 1024
_NW = 32
_TPW = 256
_L = 16


def _probs_body(x_ref, w_ref, probs_ref, psum_ref, acc_ref):
    step = pl.program_id(0)
    nsteps = pl.num_programs(0)

    @pl.when(step == 0)
    def _init():
        acc_ref[...] = jnp.zeros_like(acc_ref)

    logits = lax.dot_general(
        w_ref[...], x_ref[...], (((1,), (1,)), ((), ())),
        preferred_element_type=jnp.float32)
    m = jnp.max(logits, axis=0, keepdims=True)
    e = jnp.exp(logits - m)
    probs = e / jnp.sum(e, axis=0, keepdims=True)
    for c in range(_ROWS // _TPW):
        probs_ref[c, :, :] = probs[:, c * _TPW:(c + 1) * _TPW]
    acc_ref[...] += jnp.sum(probs, axis=1, keepdims=True)

    @pl.when(step == nsteps - 1)
    def _fin():
        psum_ref[...] = jnp.broadcast_to(acc_ref[...], (_NE, _L))


def _sc_topk_body(probs_hbm, psum_hbm, gate_hbm, idx_hbm, apart_hbm, pv,
                  psum_v, gv, iv, acc_v):
    wid = lax.axis_index("s") * 2 + lax.axis_index("c")
    pltpu.sync_copy(probs_hbm.at[wid], pv)
    pltpu.sync_copy(psum_hbm, psum_v)

    neg = jnp.full((_L,), -1.0, jnp.float32)
    zero_i = jnp.zeros((_L,), jnp.int32)
    zero_f = jnp.zeros((_L,), jnp.float32)

    def group(g, acc):
        t0 = g * _L
        vals = [neg] * _K
        idxs = [zero_i] * _K
        pvs = [zero_f] * _K
        for e in range(_NE):
            p = pv[e, pl.ds(t0, _L)]
            ei = jnp.full((_L,), e, jnp.int32)
            pe = psum_v[e, pl.ds(0, _L)]
            c_prev = None
            new_vals = []
            new_idxs = []
            new_pvs = []
            for j in range(_K):
                c_j = p > vals[j]
                ins_v = jnp.where(c_j, p, vals[j])
                ins_i = jnp.where(c_j, ei, idxs[j])
                ins_p = jnp.where(c_j, pe, pvs[j])
                if c_prev is None:
                    new_vals.append(ins_v)
                    new_idxs.append(ins_i)
                    new_pvs.append(ins_p)
                else:
                    new_vals.append(jnp.where(c_prev, vals[j - 1], ins_v))
                    new_idxs.append(jnp.where(c_prev, idxs[j - 1], ins_i))
                    new_pvs.append(jnp.where(c_prev, pvs[j - 1], ins_p))
                c_prev = c_j
            vals = new_vals
            idxs = new_idxs
            pvs = new_pvs
        s = vals[0]
        for j in range(1, _K):
            s = s + vals[j]
        for j in range(_K):
            gv[j, pl.ds(t0, _L)] = vals[j] / s
            iv[j, pl.ds(t0, _L)] = idxs[j]
            acc = acc + pvs[j]
        return acc

    acc = lax.fori_loop(0, _TPW // _L, group, zero_f)
    acc_v[...] = acc

    pltpu.sync_copy(gv, gate_hbm.at[wid])
    pltpu.sync_copy(iv, idx_hbm.at[wid])
    pltpu.sync_copy(acc_v, apart_hbm.at[wid])


def _aux_body(apart_ref, aux_ref, *, n_tokens):
    scale = _NE / (float(n_tokens) * float(n_tokens))
    aux_ref[...] = (scale * jnp.sum(apart_ref[...], keepdims=True)
                    ).reshape(1, 1)


def kernel(x, W):
    b, s, d = x.shape
    n = b * s
    xf = x.reshape(n, d)
    grid = n // _ROWS

    probs, psum = pl.pallas_call(
        _probs_body,
        grid=(grid,),
        in_specs=[
            pl.BlockSpec((_ROWS, d), lambda i: (i, 0)),
            pl.BlockSpec((_NE, d), lambda i: (0, 0)),
        ],
        out_specs=[
            pl.BlockSpec((_ROWS // _TPW, _NE, _TPW), lambda i: (i, 0, 0)),
            pl.BlockSpec((_NE, _L), lambda i: (0, 0)),
        ],
        out_shape=[
            jax.ShapeDtypeStruct((_NW, _NE, _TPW), jnp.float32),
            jax.ShapeDtypeStruct((_NE, _L), jnp.float32),
        ],
        scratch_shapes=[pltpu.VMEM((_NE, 1), jnp.float32)],
        compiler_params=pltpu.CompilerParams(
            dimension_semantics=("arbitrary",)),
    )(xf, W)

    mesh = plsc.VectorSubcoreMesh(core_axis_name="c", subcore_axis_name="s")
    gate3, idx3, apart = functools.partial(
        pl.kernel,
        mesh=mesh,
        out_type=[
            jax.ShapeDtypeStruct((_NW, _K, _TPW), jnp.float32),
            jax.ShapeDtypeStruct((_NW, _K, _TPW), jnp.int32),
            jax.ShapeDtypeStruct((_NW, _L), jnp.float32),
        ],
        scratch_types=[
            pltpu.VMEM((_NE, _TPW), jnp.float32),
            pltpu.VMEM((_NE, _L), jnp.float32),
            pltpu.VMEM((_K, _TPW), jnp.float32),
            pltpu.VMEM((_K, _TPW), jnp.int32),
            pltpu.VMEM((_L,), jnp.float32),
        ],
    )(_sc_topk_body)(probs, psum)

    aux = pl.pallas_call(
        functools.partial(_aux_body, n_tokens=n),
        out_shape=jax.ShapeDtypeStruct((1, 1), jnp.float32),
    )(apart)

    gate = jnp.transpose(gate3, (0, 2, 1)).reshape(n, _K)
    idx = jnp.transpose(idx3, (0, 2, 1)).reshape(n, _K)
    return gate.astype(x.dtype), idx, aux.reshape(())


# speedup vs baseline: 1.7536x; 1.7536x over previous
import functools

import jax
import jax.numpy as jnp
from jax.experimental import pallas as pl
from jax.experimental.pallas import tpu as pltpu

_NUM_EXPERTS = 64
_TOP_K = 8
_ROWS = 1024


def _router_body(x_ref, w_ref, gate_ref, idx_ref, aux_ref, psum_ref, fsum_ref,
                 *, n_tokens):
    step = pl.program_id(0)
    nsteps = pl.num_programs(0)

    @pl.when(step == 0)
    def _init():
        psum_ref[...] = jnp.zeros_like(psum_ref)
        fsum_ref[...] = jnp.zeros_like(fsum_ref)

    x = x_ref[...]
    w = w_ref[...]
    logits = jax.lax.dot_general(
        w, x, (((1,), (1,)), ((), ())), preferred_element_type=jnp.float32)

    m = jnp.max(logits, axis=0, keepdims=True)
    e = jnp.exp(logits - m)
    s = jnp.sum(e, axis=0, keepdims=True)
    probs = e / s

    iota = jax.lax.broadcasted_iota(jnp.int32, probs.shape, 0)
    p = probs
    vals = []
    idxs = []
    for _ in range(_TOP_K):
        mv = jnp.max(p, axis=0, keepdims=True)
        ij = jnp.min(jnp.where(p == mv, iota, _NUM_EXPERTS), axis=0,
                     keepdims=True)
        vals.append(mv)
        idxs.append(ij)
        p = jnp.where(iota == ij, -1.0, p)
    v = jnp.concatenate(vals, axis=0)
    gate_ref[...] = v / jnp.sum(v, axis=0, keepdims=True)
    idx_ref[...] = jnp.concatenate(idxs, axis=0)

    mask = (p < 0).astype(jnp.float32)
    psum_ref[...] += jnp.sum(probs, axis=1, keepdims=True)
    fsum_ref[...] += jnp.sum(mask, axis=1, keepdims=True)

    @pl.when(step == nsteps - 1)
    def _finalize():
        f = fsum_ref[...] / n_tokens
        pbar = psum_ref[...] / n_tokens
        aux_ref[...] = jnp.sum(_NUM_EXPERTS * f * pbar, keepdims=True
                               ).reshape(1, 1)


def kernel(x, W):
    b, s, d = x.shape
    n = b * s
    xf = x.reshape(n, d)
    grid = n // _ROWS
    gate_t, idx_t, aux = pl.pallas_call(
        functools.partial(_router_body, n_tokens=n),
        grid=(grid,),
        in_specs=[
            pl.BlockSpec((_ROWS, d), lambda i: (i, 0)),
            pl.BlockSpec((_NUM_EXPERTS, d), lambda i: (0, 0)),
        ],
        out_specs=[
            pl.BlockSpec((_TOP_K, _ROWS), lambda i: (0, i)),
            pl.BlockSpec((_TOP_K, _ROWS), lambda i: (0, i)),
            pl.BlockSpec((1, 1), lambda i: (0, 0)),
        ],
        out_shape=[
            jax.ShapeDtypeStruct((_TOP_K, n), jnp.float32),
            jax.ShapeDtypeStruct((_TOP_K, n), jnp.int32),
            jax.ShapeDtypeStruct((1, 1), jnp.float32),
        ],
        scratch_shapes=[
            pltpu.VMEM((_NUM_EXPERTS, 1), jnp.float32),
            pltpu.VMEM((_NUM_EXPERTS, 1), jnp.float32),
        ],
        compiler_params=pltpu.CompilerParams(
            dimension_semantics=("arbitrary",)),
    )(xf, W)
    return gate_t.T.astype(x.dtype), idx_t.T, aux.reshape(())
